# Initial kernel scaffold; baseline (speedup 1.0000x reference)
import jax
import jax.numpy as jnp
from jax.experimental import pallas as pl


def kernel(node_features, edge_features, edge_index, W1e, b1e, W2e, b2e, W1n, b1n, W2n, b2n):
    raise NotImplementedError("write your pallas kernel here")


# speedup vs baseline: 2.4943x; 2.4943x over previous
import functools

import jax
import jax.numpy as jnp
from jax import lax
from jax.experimental import pallas as pl
from jax.experimental.pallas import tpu as pltpu
from jax.experimental.pallas import tpu_sc as plsc

N = 10000
E = 320000
D = 128
EDGE_DIM = 16

NC = 2
NS = 16
NW = NC * NS
CHUNK = 512
SUB = 128
EPW = 10240
EPAD = EPW * NW
ISTEP = 1024
STEPS = EPW // ISTEP
HALVES = ISTEP // CHUNK
GPC = CHUNK // SUB
SCHUNK = 256
SHALVES = ISTEP // SCHUNK
SGPC = SCHUNK // SUB
NPAD = 240
NACC = N + NPAD
ZROWS = NACC // NS

@functools.lru_cache(maxsize=None)
def _sc_mesh():
    return plsc.VectorSubcoreMesh(core_axis_name="c", subcore_axis_name="s",
                                  num_cores=NC, num_subcores=NS)


def _gather_body(nf_hbm, src_hbm, dst_hbm, src_out, dst_out, idx_v, rows_v, sem):
    c = lax.axis_index("c")
    s = lax.axis_index("s")
    wid = s * NC + c
    base_g = wid * (EPW // SUB)

    def do_table(idx2d_hbm, out_hbm):
        def step(i, carry):
            g0 = base_g + i * (ISTEP // SUB)
            pltpu.sync_copy(idx2d_hbm.at[pl.ds(g0, ISTEP // SUB)], idx_v)
            for h in range(HALVES):
                for j in range(GPC):
                    pltpu.async_copy(
                        nf_hbm.at[idx_v.at[h * GPC + j]],
                        rows_v.at[pl.ds(j * SUB, SUB)],
                        sem,
                    ).wait()
                pltpu.sync_copy(
                    rows_v, out_hbm.at[pl.ds((g0 + h * GPC) * SUB, CHUNK)])
            return carry

        lax.fori_loop(0, STEPS, step, 0)

    do_table(src_hbm, src_out)
    do_table(dst_hbm, dst_out)


@functools.lru_cache(maxsize=None)
def _gather_call():
    return pl.kernel(
        _gather_body,
        out_type=(
            jax.ShapeDtypeStruct((EPAD, D), jnp.float32),
            jax.ShapeDtypeStruct((EPAD, D), jnp.float32),
        ),
        mesh=_sc_mesh(),
        scratch_types=[
            pltpu.VMEM((ISTEP // SUB, SUB), jnp.int32),
            pltpu.VMEM((CHUNK, D), jnp.float32),
            pltpu.SemaphoreType.DMA,
        ],
    )


def _scatter_body(msg_hbm, dst_hbm, zeros_hbm, part_out, acc_sh, idx_v, rows_v):
    c = lax.axis_index("c")
    s = lax.axis_index("s")
    pltpu.sync_copy(zeros_hbm.at[pl.ds(s * ZROWS, ZROWS)],
                    acc_sh.at[pl.ds(s * ZROWS, ZROWS)])
    plsc.subcore_barrier()

    base_g = (c * NS + s) * (EPW // SUB)

    def step(i, carry):
        g0 = base_g + i * (ISTEP // SUB)
        pltpu.sync_copy(dst_hbm.at[pl.ds(g0, ISTEP // SUB)], idx_v)
        for h in range(SHALVES):
            pltpu.sync_copy(
                msg_hbm.at[pl.ds((g0 + h * SGPC) * SUB, SCHUNK)], rows_v)
            for j in range(SGPC):
                pltpu.sync_copy(
                    rows_v.at[pl.ds(j * SUB, SUB)],
                    acc_sh.at[idx_v.at[h * SGPC + j]],
                    add=True,
                )
        return carry

    lax.fori_loop(0, STEPS, step, 0)
    plsc.subcore_barrier()
    pltpu.sync_copy(acc_sh.at[pl.ds(s * ZROWS, ZROWS)],
                    part_out.at[c].at[pl.ds(s * ZROWS, ZROWS)])


@functools.lru_cache(maxsize=None)
def _scatter_call():
    return pl.kernel(
        _scatter_body,
        out_type=jax.ShapeDtypeStruct((NC, NACC, D), jnp.float32),
        mesh=_sc_mesh(),
        scratch_types=[
            pltpu.VMEM_SHARED((NACC, D), jnp.float32),
            pltpu.VMEM((ISTEP // SUB, SUB), jnp.int32),
            pltpu.VMEM((SCHUNK, D), jnp.float32),
        ],
    )


BE = 1024


def _mlp_body(src_ref, dst_ref, ef_ref, w1s_ref, w1d_ref, w1f_ref, b1e_ref,
              w2e_ref, b2e_ref, w1nd_ref, w1nm_ref, b1n_ref, w2n_ref, b2n_ref,
              out_ref):
    src = src_ref[...]
    dst = dst_ref[...]
    ef = ef_ref[...]
    h = src @ w1s_ref[...] + dst @ w1d_ref[...] + ef @ w1f_ref[...] + b1e_ref[...]
    h = jnp.maximum(h, 0.0)
    msg = jnp.maximum(h @ w2e_ref[...] + b2e_ref[...], 0.0)
    g = dst @ w1nd_ref[...] + msg @ w1nm_ref[...] + b1n_ref[...]
    g = jnp.maximum(g, 0.0)
    out_ref[...] = jnp.maximum(g @ w2n_ref[...] + b2n_ref[...], 0.0)


def _mlp_call(src_feat, dst_feat, ef, w1s, w1d, w1f, b1e, w2e, b2e,
              w1nd, w1nm, b1n, w2n, b2n):
    grid = (EPAD // BE,)
    eb = lambda i: (i, 0)
    wb = lambda i: (0, 0)
    return pl.pallas_call(
        _mlp_body,
        grid=grid,
        in_specs=[
            pl.BlockSpec((BE, D), eb),
            pl.BlockSpec((BE, D), eb),
            pl.BlockSpec((BE, EDGE_DIM), eb),
            pl.BlockSpec((D, 32), wb),
            pl.BlockSpec((D, 32), wb),
            pl.BlockSpec((EDGE_DIM, 32), wb),
            pl.BlockSpec((1, 32), wb),
            pl.BlockSpec((32, D), wb),
            pl.BlockSpec((1, D), wb),
            pl.BlockSpec((D, 64), wb),
            pl.BlockSpec((D, 64), wb),
            pl.BlockSpec((1, 64), wb),
            pl.BlockSpec((64, D), wb),
            pl.BlockSpec((1, D), wb),
        ],
        out_specs=pl.BlockSpec((BE, D), eb),
        out_shape=jax.ShapeDtypeStruct((EPAD, D), jnp.float32),
    )(src_feat, dst_feat, ef, w1s, w1d, w1f, b1e, w2e, b2e,
      w1nd, w1nm, b1n, w2n, b2n)


BN = 1000


def _combine_body(nf_ref, pa_ref, pb_ref, out_ref):
    out_ref[...] = nf_ref[...] + pa_ref[0] + pb_ref[0]


def _combine_call(nf, parts):
    grid = (N // BN,)
    return pl.pallas_call(
        _combine_body,
        grid=grid,
        in_specs=[
            pl.BlockSpec((BN, D), lambda i: (i, 0)),
            pl.BlockSpec((1, BN, D), lambda i: (0, i, 0)),
            pl.BlockSpec((1, BN, D), lambda i: (1, i, 0)),
        ],
        name="combine",
        out_specs=pl.BlockSpec((BN, D), lambda i: (i, 0)),
        out_shape=jax.ShapeDtypeStruct((N, D), jnp.float32),
    )(nf, parts, parts)


def kernel(node_features, edge_features, edge_index, W1e, b1e, W2e, b2e,
           W1n, b1n, W2n, b2n):
    src = edge_index[0].astype(jnp.int32)
    dst = edge_index[1].astype(jnp.int32)
    pad = EPAD - E
    ar = jnp.arange(pad, dtype=jnp.int32)
    pad_gather = ar % N
    pad_scatter = N + (ar % NPAD)

    src2d = jnp.concatenate([src, pad_gather]).reshape(EPAD // SUB, SUB)
    dstg2d = jnp.concatenate([dst, pad_gather]).reshape(EPAD // SUB, SUB)
    dsts2d = jnp.concatenate([dst, pad_scatter]).reshape(EPAD // SUB, SUB)
    ef_pad = jnp.concatenate(
        [edge_features, jnp.zeros((pad, EDGE_DIM), jnp.float32)])
    zeros = jnp.zeros((NACC, D), jnp.float32)

    src_feat, dst_feat = _gather_call()(node_features, src2d, dstg2d)

    msgs = _mlp_call(
        src_feat, dst_feat, ef_pad,
        W1e[:D], W1e[D:2 * D], W1e[2 * D:], b1e.reshape(1, 32),
        W2e, b2e.reshape(1, D),
        W1n[:D], W1n[D:], b1n.reshape(1, 64),
        W2n, b2n.reshape(1, D),
    )

    parts = _scatter_call()(msgs, dsts2d, zeros)
    return _combine_call(node_features, parts)
